# Initial kernel scaffold; baseline (speedup 1.0000x reference)
import jax
import jax.numpy as jnp
from jax.experimental import pallas as pl


def kernel(x, pos, batch, W1, b1, W2, b2):
    raise NotImplementedError("write your pallas kernel here")


# speedup vs baseline: 2.1992x; 2.1992x over previous
import functools

import jax
import jax.numpy as jnp
from jax.experimental import pallas as pl
from jax.experimental.pallas import tpu as pltpu

_RATIO = 0.25
_R = 0.1
_K = 32


def _fps_body(px_ref, py_ref, pz_ref, idx_ref, dist_ref, *, n, m):
    rows = px_ref.shape[0]
    lin = (jax.lax.broadcasted_iota(jnp.int32, (rows, 128), 0) * 128
           + jax.lax.broadcasted_iota(jnp.int32, (rows, 128), 1))
    valid = lin < n
    dist_ref[...] = jnp.where(valid, jnp.inf, -jnp.inf).astype(jnp.float32)
    idx_ref[0] = jnp.int32(0)

    def _coords(sel):
        msk = lin == sel
        sx = jnp.max(jnp.where(msk, px_ref[...], -jnp.inf))
        sy = jnp.max(jnp.where(msk, py_ref[...], -jnp.inf))
        sz = jnp.max(jnp.where(msk, pz_ref[...], -jnp.inf))
        return sx, sy, sz

    def body(i, carry):
        sx, sy, sz = carry
        dx = px_ref[...] - sx
        dy = py_ref[...] - sy
        dz = pz_ref[...] - sz
        d = (dx * dx + dy * dy) + dz * dz
        nd = jnp.minimum(dist_ref[...], d)
        dist_ref[...] = nd
        gm = jnp.max(nd)
        nxt = jnp.min(jnp.where(nd == gm, lin, jnp.int32(2**30)))
        idx_ref[i] = nxt
        return _coords(nxt)

    jax.lax.fori_loop(1, m, body, _coords(jnp.int32(0)), unroll=False)


def _fps(pos, m):
    n = pos.shape[0]
    rows = (n + 127) // 128
    pad = rows * 128 - n
    p = jnp.pad(pos, ((0, pad), (0, 0)), constant_values=1e9)
    px = p[:, 0].reshape(rows, 128)
    py = p[:, 1].reshape(rows, 128)
    pz = p[:, 2].reshape(rows, 128)
    return pl.pallas_call(
        functools.partial(_fps_body, n=n, m=m),
        out_shape=jax.ShapeDtypeStruct((m,), jnp.int32),
        out_specs=pl.BlockSpec(memory_space=pltpu.SMEM),
        scratch_shapes=[pltpu.VMEM((rows, 128), jnp.float32)],
    )(px, py, pz)


def kernel(x, pos, batch, W1, b1, W2, b2):
    n, d = x.shape
    m = int(_RATIO * n)
    indices = _fps(pos, m)

    sq = jnp.sum(pos * pos, axis=-1)
    allidx = jnp.arange(n)
    chunk = 2000
    nbrs, dks = [], []
    for s in range(0, n, chunk):
        e = min(s + chunk, n)
        d2 = sq[s:e, None] + sq[None, :] - 2.0 * (pos[s:e] @ pos.T)
        d2 = jnp.maximum(d2, 0.0)
        same = batch[s:e, None] == batch[None, :]
        selfm = allidx[s:e, None] == allidx[None, :]
        d2 = jnp.where(same & (~selfm), d2, jnp.inf)
        negd, idx = jax.lax.top_k(-d2, _K)
        nbrs.append(idx)
        dks.append(-negd)
    nbr = jnp.concatenate(nbrs, axis=0)
    dk = jnp.concatenate(dks, axis=0)
    valid = dk <= _R * _R

    W1a, W1b = W1[:d], W1[d:]
    g = x @ W1a + pos @ W1b + b1
    p1 = pos @ W1b
    gn = jnp.where(valid[:, :, None], g[nbr], -jnp.inf)
    M = jnp.max(gn, axis=1)
    agg = jnp.maximum(M - p1, 0.0)
    out = jnp.maximum(agg @ W2 + b2, 0.0)

    return (out, out[indices], pos[indices], batch[indices])
